# baseline (speedup 1.0000x reference)
import functools

import jax
import jax.numpy as jnp
from jax import lax
from jax.experimental import pallas as pl
from jax.experimental.pallas import tpu as pltpu
from jax.experimental.pallas import tpu_sc as plsc

DIM = 2048
N_ROWS = 4 * 8192
NC, NS = 2, 16
NW = NC * NS
C = 16
NBUF = 3


def _make_sc_body(rows_per_worker):
    n_chunks = rows_per_worker // C
    iters = (n_chunks - 1) // NBUF
    tail_first = iters * NBUF

    def _gather_body(table_hbm, idx_hbm, out_hbm, idx_v, *rest):
        bufs = rest[:NBUF]
        gsems = rest[NBUF:2 * NBUF]
        osems = rest[2 * NBUF:]

        wid = lax.axis_index("s") * NC + lax.axis_index("c")
        base = wid * rows_per_worker

        pltpu.sync_copy(idx_hbm.at[pl.ds(base, rows_per_worker)], idx_v)

        def gather(c, i):
            return pltpu.async_copy(
                table_hbm.at[idx_v.at[pl.ds(c * C, C)]], bufs[i], gsems[i])

        def fire_out(c, i):
            pltpu.async_copy(
                bufs[i], out_hbm.at[pl.ds(base + c * C, C)], osems[i])

        def wait_gather(c, i):
            pltpu.make_async_copy(
                table_hbm.at[idx_v.at[pl.ds(c * C, C)]], bufs[i],
                gsems[i]).wait()

        def wait_out(i):
            pltpu.make_async_copy(
                bufs[i], out_hbm.at[pl.ds(0, C)], osems[i]).wait()

        def body(t, carry):
            for b in range(NBUF):
                c = t * NBUF + b

                @pl.when(t > 0)
                def _drain(b=b):
                    wait_out(b)

                gather(c, b)
                j = (b - 1) % NBUF
                if b == 0:
                    @pl.when(t > 0)
                    def _lagged():
                        wait_gather(t * NBUF - 1, j)
                        fire_out(t * NBUF - 1, j)
                else:
                    wait_gather(c - 1, j)
                    fire_out(c - 1, j)
            return carry

        lax.fori_loop(0, iters, body, 0)

        last = tail_first - 1
        wait_gather(last, last % NBUF)
        fire_out(last, last % NBUF)
        for c in range(tail_first, n_chunks):
            i = c % NBUF
            wait_out(i)
            gather(c, i).wait()
            fire_out(c, i)

        for i in range(NBUF):
            wait_out(i)

    return _gather_body


@functools.lru_cache(maxsize=1)
def _build_sc_gather():
    rw = N_ROWS // NW
    mesh = plsc.VectorSubcoreMesh(
        core_axis_name="c", subcore_axis_name="s",
        num_cores=NC, num_subcores=NS)
    return pl.kernel(
        _make_sc_body(rw),
        out_type=jax.ShapeDtypeStruct((N_ROWS, DIM), jnp.float32),
        mesh=mesh,
        scratch_types=(
            [pltpu.VMEM((rw,), jnp.int32)]
            + [pltpu.VMEM((C, DIM), jnp.float32) for _ in range(NBUF)]
            + [pltpu.SemaphoreType.DMA for _ in range(2 * NBUF)]
        ),
    )


def kernel(position_ids, pe):
    idx = position_ids.reshape(N_ROWS)
    out = _build_sc_gather()(pe, idx)
    return out.reshape(position_ids.shape + (DIM,))


# speedup vs baseline: 1.0081x; 1.0081x over previous
import functools

import jax
import jax.numpy as jnp
from jax import lax
from jax.experimental import pallas as pl
from jax.experimental.pallas import tpu as pltpu
from jax.experimental.pallas import tpu_sc as plsc

DIM = 2048
N_ROWS = 4 * 8192
NC, NS = 2, 16
NW = NC * NS
C = 8
NBUF = 4


def _make_sc_body(rows_per_worker):
    supersteps = rows_per_worker // (NBUF * C)

    def _gather_body(table_hbm, idx_hbm, out_hbm, idx_v, *rest):
        bufs = rest[:NBUF]
        gsems = rest[NBUF:2 * NBUF]
        osems = rest[2 * NBUF:]

        wid = lax.axis_index("s") * NC + lax.axis_index("c")
        base = wid * rows_per_worker

        pltpu.sync_copy(idx_hbm.at[pl.ds(base, rows_per_worker)], idx_v)

        def superstep(t, carry):
            for grp in range(2):
                gdescs = []
                for b in range(2):
                    i = grp * 2 + b
                    row0 = (t * NBUF + i) * C

                    @pl.when(t > 0)
                    def _drain(i=i):
                        pltpu.make_async_copy(
                            bufs[i], out_hbm.at[pl.ds(0, C)], osems[i]).wait()

                    gdescs.append(pltpu.async_copy(
                        table_hbm.at[idx_v.at[pl.ds(row0, C)]],
                        bufs[i], gsems[i]))
                for b in range(2):
                    i = grp * 2 + b
                    row0 = (t * NBUF + i) * C
                    gdescs[b].wait()
                    pltpu.async_copy(
                        bufs[i], out_hbm.at[pl.ds(base + row0, C)], osems[i])
            return carry

        lax.fori_loop(0, supersteps, superstep, 0)

        for i in range(NBUF):
            pltpu.make_async_copy(
                bufs[i], out_hbm.at[pl.ds(0, C)], osems[i]).wait()

    return _gather_body


@functools.lru_cache(maxsize=1)
def _build_sc_gather():
    rw = N_ROWS // NW
    mesh = plsc.VectorSubcoreMesh(
        core_axis_name="c", subcore_axis_name="s",
        num_cores=NC, num_subcores=NS)
    return pl.kernel(
        _make_sc_body(rw),
        out_type=jax.ShapeDtypeStruct((N_ROWS, DIM), jnp.float32),
        mesh=mesh,
        scratch_types=(
            [pltpu.VMEM((rw,), jnp.int32)]
            + [pltpu.VMEM((C, DIM), jnp.float32) for _ in range(NBUF)]
            + [pltpu.SemaphoreType.DMA for _ in range(2 * NBUF)]
        ),
    )


def kernel(position_ids, pe):
    idx = position_ids.reshape(N_ROWS)
    out = _build_sc_gather()(pe, idx)
    return out.reshape(position_ids.shape + (DIM,))
